# baseline (speedup 1.0000x reference)
import functools

import jax
import jax.numpy as jnp
from jax import lax
from jax.experimental import pallas as pl
from jax.experimental.pallas import tpu as pltpu
from jax.experimental.pallas import tpu_sc as plsc


@functools.cache
def _build(n, vocab, d):
    info = plsc.get_sparse_core_info()
    nw = info.num_cores * info.num_subcores
    ch = 128
    n_per_w = n // nw
    nch = n_per_w // ch
    assert n == nw * nch * ch

    mesh = plsc.VectorSubcoreMesh(core_axis_name="c", subcore_axis_name="s")

    @functools.partial(
        pl.kernel,
        mesh=mesh,
        out_type=jax.ShapeDtypeStruct((n, d), jnp.float32),
        scratch_types=[
            pltpu.VMEM((nch, ch), jnp.int32),
            pltpu.VMEM((ch, d), jnp.float32),
            pltpu.SemaphoreType.DMA,
        ],
        compiler_params=pltpu.CompilerParams(use_tc_tiling_on_sc=False),
    )
    def run(table_hbm, idx_hbm, out_hbm, idx_v, rows_v, sem):
        wid = lax.axis_index("s") * info.num_cores + lax.axis_index("c")
        base = wid * n_per_w
        pltpu.sync_copy(idx_hbm.at[wid], idx_v)

        def body(j, carry):
            pltpu.async_copy(table_hbm.at[idx_v.at[j]], rows_v, sem).wait()
            pltpu.sync_copy(rows_v, out_hbm.at[pl.ds(base + j * ch, ch)])
            return carry

        lax.fori_loop(0, nch, body, 0)

    return run, nw, nch, ch


def kernel(input_ids, table):
    seq, batch = input_ids.shape
    vocab, d = table.shape
    n = seq * batch
    run, nw, nch, ch = _build(n, vocab, d)
    idx3 = input_ids.reshape(nw, nch, ch)
    out = run(table, idx3)
    return out.reshape(seq, batch, d)


# speedup vs baseline: 1.1148x; 1.1148x over previous
import functools

import jax
import jax.numpy as jnp
from jax import lax
from jax.experimental import pallas as pl
from jax.experimental.pallas import tpu as pltpu
from jax.experimental.pallas import tpu_sc as plsc


@functools.cache
def _build(n, vocab, d):
    info = plsc.get_sparse_core_info()
    nw = info.num_cores * info.num_subcores
    ch = 128
    kk = 4
    blk = ch * kk
    n_per_w = n // nw
    nch = n_per_w // ch
    nblk = n_per_w // blk
    assert n == nw * nch * ch and nblk % 2 == 0

    mesh = plsc.VectorSubcoreMesh(core_axis_name="c", subcore_axis_name="s")

    @functools.partial(
        pl.kernel,
        mesh=mesh,
        out_type=jax.ShapeDtypeStruct((n, d), jnp.float32),
        scratch_types=[
            pltpu.VMEM((nch, ch), jnp.int32),
            pltpu.VMEM((2, blk, d), jnp.float32),
            pltpu.SemaphoreType.DMA,
            pltpu.SemaphoreType.DMA,
            pltpu.SemaphoreType.DMA,
            pltpu.SemaphoreType.DMA,
        ],
        compiler_params=pltpu.CompilerParams(use_tc_tiling_on_sc=False),
    )
    def run(table_hbm, idx_hbm, out_hbm, idx_v, rows_v, g0, g1, o0, o1):
        gsem = (g0, g1)
        osem = (o0, o1)
        wid = lax.axis_index("s") * info.num_cores + lax.axis_index("c")
        base = wid * n_per_w
        pltpu.sync_copy(idx_hbm.at[wid], idx_v)

        def fire_block(g, s):
            for b in range(kk):
                j = g * kk + b
                pltpu.async_copy(
                    table_hbm.at[idx_v.at[j]],
                    rows_v.at[s, pl.ds(b * ch, ch)],
                    gsem[s],
                )

        def drain_gathers(s):
            for b in range(kk):
                pltpu.make_async_copy(
                    table_hbm.at[idx_v.at[0]],
                    rows_v.at[s, pl.ds(b * ch, ch)],
                    gsem[s],
                ).wait()

        def wait_out(s):
            pltpu.make_async_copy(
                rows_v.at[s],
                out_hbm.at[pl.ds(base, blk)],
                osem[s],
            ).wait()

        fire_block(0, 0)

        def body(g2, carry):
            for s in range(2):
                g = g2 * 2 + s
                drain_gathers(s)
                pltpu.async_copy(
                    rows_v.at[s],
                    out_hbm.at[pl.ds(base + g * blk, blk)],
                    osem[s],
                )

                @pl.when(g >= 1)
                def _():
                    wait_out(1 - s)

                @pl.when(g + 1 < nblk)
                def _():
                    fire_block(g + 1, 1 - s)

            return carry

        lax.fori_loop(0, nblk // 2, body, 0)
        wait_out(1)

    return run, nw, nch, ch


def kernel(input_ids, table):
    seq, batch = input_ids.shape
    vocab, d = table.shape
    n = seq * batch
    run, nw, nch, ch = _build(n, vocab, d)
    idx3 = input_ids.reshape(nw, nch, ch)
    out = run(table, idx3)
    return out.reshape(seq, batch, d)
